# baseline (speedup 1.0000x reference)
import jax
import jax.numpy as jnp
from jax.experimental import pallas as pl
from jax.experimental.pallas import tpu as pltpu

_N_CLASSES = 1000
_B, _T = 4096, 26
_BLOCK = 64


def _onehot_block(idx_ref, out_ref):
    idx = idx_ref[...]
    iota = jax.lax.broadcasted_iota(jnp.int32, (_BLOCK, _T, _N_CLASSES), 2)
    out_ref[...] = (iota == idx[:, :, None]).astype(jnp.float32)


def kernel(input):
    return pl.pallas_call(
        _onehot_block,
        grid=(_B // _BLOCK,),
        in_specs=[pl.BlockSpec((_BLOCK, _T), lambda i: (i, 0))],
        out_specs=pl.BlockSpec((_BLOCK, _T, _N_CLASSES), lambda i: (i, 0, 0)),
        out_shape=jax.ShapeDtypeStruct((_B, _T, _N_CLASSES), jnp.float32),
        compiler_params=pltpu.CompilerParams(
            dimension_semantics=("parallel",),
        ),
    )(input.astype(jnp.int32))


# speedup vs baseline: 1.0012x; 1.0012x over previous
import jax
import jax.numpy as jnp
from jax.experimental import pallas as pl
from jax.experimental.pallas import tpu as pltpu

_N_CLASSES = 1000
_B, _T = 4096, 26
_BLOCK = 32
_NBUF = 8
_NSTEPS = _B // _BLOCK


def _onehot_body(idx_ref, hbm_ref, scratch_ref, sem_ref):
    i = pl.program_id(0)
    slot = jax.lax.rem(i, _NBUF)

    @pl.when(i >= _NBUF)
    def _wait_prev():
        j = i - _NBUF
        pltpu.make_async_copy(
            scratch_ref.at[slot],
            hbm_ref.at[pl.ds(j * _BLOCK, _BLOCK)],
            sem_ref.at[slot],
        ).wait()

    idx = idx_ref[...]
    iota = jax.lax.broadcasted_iota(jnp.int32, (_BLOCK, _T, _N_CLASSES), 2)
    scratch_ref[slot] = (iota == idx[:, :, None]).astype(jnp.float32)

    pltpu.make_async_copy(
        scratch_ref.at[slot],
        hbm_ref.at[pl.ds(i * _BLOCK, _BLOCK)],
        sem_ref.at[slot],
    ).start()

    @pl.when(i == _NSTEPS - 1)
    def _drain():
        for k in range(_NBUF):
            j = _NSTEPS - _NBUF + k
            pltpu.make_async_copy(
                scratch_ref.at[k],
                hbm_ref.at[pl.ds(j * _BLOCK, _BLOCK)],
                sem_ref.at[k],
            ).wait()


def kernel(input):
    return pl.pallas_call(
        _onehot_body,
        grid=(_NSTEPS,),
        in_specs=[pl.BlockSpec((_BLOCK, _T), lambda i: (i, 0))],
        out_specs=pl.BlockSpec(memory_space=pl.ANY),
        out_shape=jax.ShapeDtypeStruct((_B, _T, _N_CLASSES), jnp.float32),
        scratch_shapes=[
            pltpu.VMEM((_NBUF, _BLOCK, _T, _N_CLASSES), jnp.float32),
            pltpu.SemaphoreType.DMA((_NBUF,)),
        ],
    )(input.astype(jnp.int32))


# speedup vs baseline: 4.6507x; 4.6451x over previous
import jax
import jax.numpy as jnp
from jax.experimental import pallas as pl

_N_CLASSES = 1000
_B, _T = 4096, 26
_B_BLK = 1024


def _onehot_block(idx_ref, out_ref):
    idx = idx_ref[...]
    iota = jax.lax.broadcasted_iota(jnp.int32, (1, _N_CLASSES, _B_BLK), 1)
    out_ref[...] = (iota == idx).astype(jnp.float32)


def kernel(input):
    idx_t = input.astype(jnp.int32).T.reshape(_T, 1, _B)
    out = pl.pallas_call(
        _onehot_block,
        grid=(_T, _B // _B_BLK),
        in_specs=[pl.BlockSpec((1, 1, _B_BLK), lambda t, j: (t, 0, j))],
        out_specs=pl.BlockSpec((1, _N_CLASSES, _B_BLK), lambda t, j: (t, 0, j)),
        out_shape=jax.ShapeDtypeStruct((_T, _N_CLASSES, _B), jnp.float32),
    )(idx_t)
    return out.transpose(2, 0, 1)


# speedup vs baseline: 4.7322x; 1.0175x over previous
import jax
import jax.numpy as jnp
from jax.experimental import pallas as pl

_N_CLASSES = 1000
_B, _T = 4096, 26
_B_BLK = 2048


def _onehot_block(idx_ref, out_ref):
    idx = idx_ref[...]
    iota = jax.lax.broadcasted_iota(jnp.int32, (1, _N_CLASSES, _B_BLK), 1)
    out_ref[...] = (iota == idx).astype(jnp.float32)


def kernel(input):
    idx_t = input.astype(jnp.int32).T.reshape(_T, 1, _B)
    out = pl.pallas_call(
        _onehot_block,
        grid=(_T, _B // _B_BLK),
        in_specs=[pl.BlockSpec((1, 1, _B_BLK), lambda t, j: (t, 0, j))],
        out_specs=pl.BlockSpec((1, _N_CLASSES, _B_BLK), lambda t, j: (t, 0, j)),
        out_shape=jax.ShapeDtypeStruct((_T, _N_CLASSES, _B), jnp.float32),
    )(idx_t)
    return out.transpose(2, 0, 1)
